# baseline (speedup 1.0000x reference)
import jax
import jax.numpy as jnp
from jax.experimental import pallas as pl

_BLOCK = 1024
_N_CODES = 512
_EMBED = 32
_N_LAYERS = 3


def _fused(x_ref, cbt_ref, cbs_ref, cbsq_ref,
           w0, b0, w1, b1, w2, b2, w3, b3,
           d0, e0, d1, e1, d2, e2, d3, e3,
           xhat_ref, ids_ref):
    f32 = jnp.float32
    h = x_ref[...]
    h = jnp.maximum(jnp.dot(h, w0[...], preferred_element_type=f32) + b0[...], 0.0)
    h = jnp.maximum(jnp.dot(h, w1[...], preferred_element_type=f32) + b1[...], 0.0)
    h = jnp.maximum(jnp.dot(h, w2[...], preferred_element_type=f32) + b2[...], 0.0)
    z = jnp.dot(h, w3[...], preferred_element_type=f32) + b3[...]

    res = z
    q_sum = jnp.zeros_like(z)
    idx_cols = []
    for l in range(_N_LAYERS):
        d = (jnp.sum(res * res, axis=-1, keepdims=True)
             - 2.0 * jnp.dot(res, cbt_ref[l], preferred_element_type=f32)
             + cbsq_ref[l])
        idx = jnp.argmin(d, axis=-1)
        onehot = (jax.lax.broadcasted_iota(jnp.int32, d.shape, 1)
                  == idx[:, None]).astype(f32)
        q3 = jnp.dot(onehot, cbs_ref[l], preferred_element_type=f32)
        q = (q3[:, :_EMBED] + q3[:, _EMBED:2 * _EMBED]) + q3[:, 2 * _EMBED:]
        q_sum = q_sum + q
        res = res - q
        idx_cols.append(idx[:, None])

    h = q_sum
    h = jnp.maximum(jnp.dot(h, d0[...], preferred_element_type=f32) + e0[...], 0.0)
    h = jnp.maximum(jnp.dot(h, d1[...], preferred_element_type=f32) + e1[...], 0.0)
    h = jnp.maximum(jnp.dot(h, d2[...], preferred_element_type=f32) + e2[...], 0.0)
    xhat_ref[...] = jnp.dot(h, d3[...], preferred_element_type=f32) + e3[...]

    ids_ref[...] = jnp.concatenate(idx_cols, axis=1)


def kernel(x, codebooks, enc_W0, enc_b0, enc_W1, enc_b1, enc_W2, enc_b2,
           enc_W3, enc_b3, dec_W0, dec_b0, dec_W1, dec_b1, dec_W2, dec_b2,
           dec_W3, dec_b3):
    n, in_dim = x.shape
    grid = (n // _BLOCK,)
    f32 = jnp.float32
    bf16 = jnp.bfloat16

    def _trunc16(a):
        u = jax.lax.bitcast_convert_type(a, jnp.uint32)
        return jax.lax.bitcast_convert_type(u & jnp.uint32(0xFFFF0000), f32)

    cb_hi = _trunc16(codebooks)
    r1 = codebooks - cb_hi
    cb_mid = _trunc16(r1)
    cb_lo = r1 - cb_mid
    cb_split = jnp.concatenate([cb_hi, cb_mid, cb_lo], axis=-1)
    cb_t = jnp.swapaxes(codebooks, 1, 2)
    cb_sq = jnp.sum(codebooks * codebooks, axis=-1)[:, None, :]

    def row_spec(d):
        return pl.BlockSpec((_BLOCK, d), lambda i: (i, 0))

    def full(a):
        return pl.BlockSpec(a.shape, lambda i: (0,) * a.ndim)

    encW = [enc_W0, enc_W1, enc_W2, enc_W3]
    encB = [enc_b0.reshape(1, -1), enc_b1.reshape(1, -1),
            enc_b2.reshape(1, -1), enc_b3.reshape(1, -1)]
    decW = [dec_W0, dec_W1, dec_W2, dec_W3]
    decB = [dec_b0.reshape(1, -1), dec_b1.reshape(1, -1),
            dec_b2.reshape(1, -1), dec_b3.reshape(1, -1)]

    in_specs = [row_spec(in_dim), full(cb_t), full(cb_split), full(cb_sq)]
    operands = [x, cb_t, cb_split, cb_sq]
    for W, b in zip(encW + decW, encB + decB):
        in_specs += [full(W), full(b)]
        operands += [W, b]

    x_hat, ids = pl.pallas_call(
        _fused,
        grid=grid,
        in_specs=in_specs,
        out_specs=[row_spec(in_dim),
                   pl.BlockSpec((_BLOCK, _N_LAYERS), lambda i: (i, 0))],
        out_shape=[jax.ShapeDtypeStruct((n, in_dim), jnp.float32),
                   jax.ShapeDtypeStruct((n, _N_LAYERS), jnp.int32)],
    )(*operands)
    return x_hat, ids


# speedup vs baseline: 1.0308x; 1.0308x over previous
import jax
import jax.numpy as jnp
from jax.experimental import pallas as pl

_BLOCK = 512
_N_CODES = 512
_EMBED = 32
_N_LAYERS = 3


def _fused(x_ref, cbt_ref, cbs_ref, cbsq_ref,
           w0, b0, w1, b1, w2, b2, w3, b3,
           d0, e0, d1, e1, d2, e2, d3, e3,
           xhat_ref, ids_ref):
    f32 = jnp.float32
    h = x_ref[...]
    h = jnp.maximum(jnp.dot(h, w0[...], preferred_element_type=f32) + b0[...], 0.0)
    h = jnp.maximum(jnp.dot(h, w1[...], preferred_element_type=f32) + b1[...], 0.0)
    h = jnp.maximum(jnp.dot(h, w2[...], preferred_element_type=f32) + b2[...], 0.0)
    z = jnp.dot(h, w3[...], preferred_element_type=f32) + b3[...]

    res = z
    q_sum = jnp.zeros_like(z)
    idx_cols = []
    for l in range(_N_LAYERS):
        mm = jnp.dot(res, cbt_ref[l], preferred_element_type=f32)
        d = (jnp.sum(res * res, axis=-1, keepdims=True) + mm) + cbsq_ref[l]
        dmin = jnp.min(d, axis=-1, keepdims=True)
        lane = jax.lax.broadcasted_iota(jnp.int32, d.shape, 1)
        idx = jnp.min(jnp.where(d == dmin, lane, _N_CODES), axis=-1)
        onehot = (lane == idx[:, None]).astype(f32)
        q3 = jnp.dot(onehot, cbs_ref[l], preferred_element_type=f32)
        q = (q3[:, :_EMBED] + q3[:, _EMBED:2 * _EMBED]) + q3[:, 2 * _EMBED:]
        q_sum = q_sum + q
        res = res - q
        idx_cols.append(idx[:, None])

    h = q_sum
    h = jnp.maximum(jnp.dot(h, d0[...], preferred_element_type=f32) + e0[...], 0.0)
    h = jnp.maximum(jnp.dot(h, d1[...], preferred_element_type=f32) + e1[...], 0.0)
    h = jnp.maximum(jnp.dot(h, d2[...], preferred_element_type=f32) + e2[...], 0.0)
    xhat_ref[...] = jnp.dot(h, d3[...], preferred_element_type=f32) + e3[...]

    ids_ref[...] = jnp.concatenate(idx_cols, axis=1)


def kernel(x, codebooks, enc_W0, enc_b0, enc_W1, enc_b1, enc_W2, enc_b2,
           enc_W3, enc_b3, dec_W0, dec_b0, dec_W1, dec_b1, dec_W2, dec_b2,
           dec_W3, dec_b3):
    n, in_dim = x.shape
    grid = (n // _BLOCK,)
    f32 = jnp.float32

    def _trunc16(a):
        u = jax.lax.bitcast_convert_type(a, jnp.uint32)
        return jax.lax.bitcast_convert_type(u & jnp.uint32(0xFFFF0000), f32)

    cb_hi = _trunc16(codebooks)
    r1 = codebooks - cb_hi
    cb_mid = _trunc16(r1)
    cb_lo = r1 - cb_mid
    cb_split = jnp.concatenate([cb_hi, cb_mid, cb_lo], axis=-1)
    cb_t = -2.0 * jnp.swapaxes(codebooks, 1, 2)
    cb_sq = jnp.sum(codebooks * codebooks, axis=-1)[:, None, :]

    def row_spec(d):
        return pl.BlockSpec((_BLOCK, d), lambda i: (i, 0))

    def full(a):
        return pl.BlockSpec(a.shape, lambda i: (0,) * a.ndim)

    encW = [enc_W0, enc_W1, enc_W2, enc_W3]
    encB = [enc_b0.reshape(1, -1), enc_b1.reshape(1, -1),
            enc_b2.reshape(1, -1), enc_b3.reshape(1, -1)]
    decW = [dec_W0, dec_W1, dec_W2, dec_W3]
    decB = [dec_b0.reshape(1, -1), dec_b1.reshape(1, -1),
            dec_b2.reshape(1, -1), dec_b3.reshape(1, -1)]

    in_specs = [row_spec(in_dim), full(cb_t), full(cb_split), full(cb_sq)]
    operands = [x, cb_t, cb_split, cb_sq]
    for W, b in zip(encW + decW, encB + decB):
        in_specs += [full(W), full(b)]
        operands += [W, b]

    x_hat, ids = pl.pallas_call(
        _fused,
        grid=grid,
        in_specs=in_specs,
        out_specs=[row_spec(in_dim),
                   pl.BlockSpec((_BLOCK, _N_LAYERS), lambda i: (i, 0))],
        out_shape=[jax.ShapeDtypeStruct((n, in_dim), jnp.float32),
                   jax.ShapeDtypeStruct((n, _N_LAYERS), jnp.int32)],
    )(*operands)
    return x_hat, ids
